# baseline (device time: 79573 ns/iter reference)
import jax
import jax.numpy as jnp
from jax import lax
from jax.experimental import pallas as pl
from jax.experimental.pallas import tpu as pltpu

N_DEV = 4
E_LOCAL = 8
E_TOTAL = 32
N_TOK = 2048
D = 512
H = 1024
CHUNK = N_TOK // N_DEV
HALF = CHUNK // 2


def kernel(x, router_W, route_idx, expert_W, shared_W):
    def body(x_ref, rW_ref, idx_ref, eW_ref, sW_ref, out_ref,
             pbuf, recv_r, recv_l, xb, eWb, sWb, send_sems, recv_sems):
        p = lax.axis_index("i")
        left = lax.rem(p - 1 + N_DEV, N_DEV)
        right = lax.rem(p + 1, N_DEV)

        xb[:, :] = x_ref[:, :].astype(jnp.bfloat16)
        eWb[:, :, :] = eW_ref[:, :, :].astype(jnp.bfloat16)
        sWb[:, :] = sW_ref[:, :].astype(jnp.bfloat16)

        def compute_chunk(c, add_shared=False):
            xq = xb[pl.ds(c * CHUNK, CHUNK), :]
            idxq = idx_ref[pl.ds(c * CHUNK, CHUNK), :]
            scores = jnp.dot(xq, rW_ref[:, :].astype(jnp.bfloat16),
                             preferred_element_type=jnp.float32)
            mx = jnp.max(scores, axis=1, keepdims=True)
            exs = jnp.exp(scores - mx)
            probs = exs / jnp.sum(exs, axis=1, keepdims=True)
            iota = lax.broadcasted_iota(jnp.int32, (CHUNK, E_TOTAL), 1)
            psq = jnp.sum(jnp.where(idxq == iota, probs, 0.0),
                          axis=1, keepdims=True)
            acc = None
            for j in range(E_LOCAL):
                e = p * E_LOCAL + j
                w = jnp.where(idxq == e, psq, 0.0).astype(jnp.bfloat16)
                contrib = jnp.dot(xq * w, eWb[j],
                                  preferred_element_type=jnp.float32)
                acc = contrib if acc is None else acc + contrib
            if add_shared:
                acc = acc + jnp.dot(xq, sWb[:, :],
                                    preferred_element_type=jnp.float32)
            pbuf[pl.ds(c * CHUNK, CHUNK), :] = acc.astype(jnp.bfloat16)

        def rs_rdma(s):
            sc_r = lax.rem(p - s + N_DEV, N_DEV)
            sc_l = lax.rem(p + s, N_DEV)
            r = pltpu.make_async_remote_copy(
                src_ref=pbuf.at[pl.ds(sc_r * CHUNK, HALF)],
                dst_ref=recv_r.at[s],
                send_sem=send_sems.at[s],
                recv_sem=recv_sems.at[s],
                device_id=(right,),
                device_id_type=pl.DeviceIdType.MESH,
            )
            l = pltpu.make_async_remote_copy(
                src_ref=pbuf.at[pl.ds(sc_l * CHUNK + HALF, HALF)],
                dst_ref=recv_l.at[s],
                send_sem=send_sems.at[3 + s],
                recv_sem=recv_sems.at[3 + s],
                device_id=(left,),
                device_id_type=pl.DeviceIdType.MESH,
            )
            return r, l

        def rs_add(s):
            ac_r = lax.rem(p - 1 - s + N_DEV, N_DEV)
            ac_l = lax.rem(p + 1 + s, N_DEV)
            pbuf[pl.ds(ac_r * CHUNK, HALF), :] += recv_r[s]
            pbuf[pl.ds(ac_l * CHUNK + HALF, HALF), :] += recv_l[s]

        def emit(c, half):
            rows = c * CHUNK + half * HALF
            out_ref[pl.ds(rows, HALF), :] = pbuf[pl.ds(rows, HALF), :].astype(
                jnp.float32)

        compute_chunk(p, add_shared=True)

        barrier = pltpu.get_barrier_semaphore()
        for nbr in [left, right]:
            pl.semaphore_signal(barrier, inc=1, device_id=(nbr,),
                                device_id_type=pl.DeviceIdType.MESH)
        pl.semaphore_wait(barrier, 2)

        r0, l0 = rs_rdma(0)
        r0.start()
        l0.start()
        compute_chunk(lax.rem(p + 1, N_DEV))
        compute_chunk(lax.rem(p - 1 + N_DEV, N_DEV))
        r0.wait()
        l0.wait()
        rs_add(0)

        r1, l1 = rs_rdma(1)
        r1.start()
        l1.start()
        compute_chunk(lax.rem(p + 2, N_DEV))
        r1.wait()
        l1.wait()
        rs_add(1)

        r2, l2 = rs_rdma(2)
        r2.start()
        l2.start()
        r2.wait()
        l2.wait()
        rs_add(2)

        ag = []
        for s in range(N_DEV - 1):
            gc_r = lax.rem(p + 1 - s + N_DEV, N_DEV)
            gc_l = lax.rem(p - 1 + s + N_DEV, N_DEV)
            r = pltpu.make_async_remote_copy(
                src_ref=pbuf.at[pl.ds(gc_r * CHUNK, HALF)],
                dst_ref=pbuf.at[pl.ds(gc_r * CHUNK, HALF)],
                send_sem=send_sems.at[6 + s],
                recv_sem=recv_sems.at[6 + s],
                device_id=(right,),
                device_id_type=pl.DeviceIdType.MESH,
            )
            l = pltpu.make_async_remote_copy(
                src_ref=pbuf.at[pl.ds(gc_l * CHUNK + HALF, HALF)],
                dst_ref=pbuf.at[pl.ds(gc_l * CHUNK + HALF, HALF)],
                send_sem=send_sems.at[9 + s],
                recv_sem=recv_sems.at[9 + s],
                device_id=(left,),
                device_id_type=pl.DeviceIdType.MESH,
            )
            r.start()
            l.start()
            if s == 0:
                emit(lax.rem(p + 1, N_DEV), 0)
                emit(lax.rem(p - 1 + N_DEV, N_DEV), 1)
            elif s == 1:
                emit(p, 0)
                emit(p, 1)
            else:
                emit(lax.rem(p - 1 + N_DEV, N_DEV), 0)
                emit(lax.rem(p + 1, N_DEV), 1)
            r.wait()
            l.wait()
        emit(lax.rem(p + 2, N_DEV), 0)
        emit(lax.rem(p + 2, N_DEV), 1)

    return pl.pallas_call(
        body,
        out_shape=jax.ShapeDtypeStruct((N_TOK, H), jnp.float32),
        in_specs=[pl.BlockSpec(memory_space=pltpu.VMEM)] * 5,
        out_specs=pl.BlockSpec(memory_space=pltpu.VMEM),
        scratch_shapes=[
            pltpu.VMEM((N_TOK, H), jnp.bfloat16),
            pltpu.VMEM((N_DEV - 1, HALF, H), jnp.bfloat16),
            pltpu.VMEM((N_DEV - 1, HALF, H), jnp.bfloat16),
            pltpu.VMEM((N_TOK, D), jnp.bfloat16),
            pltpu.VMEM((E_LOCAL, D, H), jnp.bfloat16),
            pltpu.VMEM((D, H), jnp.bfloat16),
            pltpu.SemaphoreType.DMA((12,)),
            pltpu.SemaphoreType.DMA((12,)),
        ],
        compiler_params=pltpu.CompilerParams(
            collective_id=0,
            vmem_limit_bytes=100 * 1024 * 1024,
        ),
    )(x, router_W, route_idx, expert_W, shared_W)


# device time: 68923 ns/iter; 1.1545x vs baseline; 1.1545x over previous
import jax
import jax.numpy as jnp
from jax import lax
from jax.experimental import pallas as pl
from jax.experimental.pallas import tpu as pltpu

N_DEV = 4
E_LOCAL = 8
E_TOTAL = 32
N_TOK = 2048
D = 512
H = 1024
CHUNK = N_TOK // N_DEV
HALF = CHUNK // 2
W = 2
WCOL = H // W


def kernel(x, router_W, route_idx, expert_W, shared_W):
    def body(x_ref, rW_ref, idx_ref, eW_ref, sW_ref, out_ref,
             pbuf, recv_r, recv_l, send_sems, recv_sems):
        p = lax.axis_index("i")
        left = lax.rem(p - 1 + N_DEV, N_DEV)
        right = lax.rem(p + 1, N_DEV)

        def routing(c):
            xq = x_ref[pl.ds(c * CHUNK, CHUNK), :]
            idxq = idx_ref[pl.ds(c * CHUNK, CHUNK), :]
            scores = jnp.dot(xq, rW_ref[:, :],
                             preferred_element_type=jnp.float32)
            mx = jnp.max(scores, axis=1, keepdims=True)
            exs = jnp.exp(scores - mx)
            probs = exs / jnp.sum(exs, axis=1, keepdims=True)
            iota = lax.broadcasted_iota(jnp.int32, (CHUNK, E_TOTAL), 1)
            psq = jnp.sum(jnp.where(idxq == iota, probs, 0.0),
                          axis=1, keepdims=True)
            return psq

        def compute_half(c, w, psq, add_shared=False):
            xq = x_ref[pl.ds(c * CHUNK, CHUNK), :]
            idxq = idx_ref[pl.ds(c * CHUNK, CHUNK), :]
            cols = slice(w * WCOL, (w + 1) * WCOL)
            acc = None
            for j in range(E_LOCAL):
                e = p * E_LOCAL + j
                wgt = jnp.where(idxq == e, psq, 0.0)
                contrib = jnp.dot(xq * wgt, eW_ref[j][:, cols],
                                  preferred_element_type=jnp.float32)
                acc = contrib if acc is None else acc + contrib
            if add_shared:
                acc = acc + jnp.dot(xq, sW_ref[:, cols],
                                    preferred_element_type=jnp.float32)
            pbuf[w, pl.ds(c * CHUNK, CHUNK), :] = acc.astype(jnp.bfloat16)

        def sem(phase, d, w, s):
            return ((phase * 2 + d) * W + w) * (N_DEV - 1) + s

        def rs_pair(w, s):
            sc_r = lax.rem(p - s + N_DEV, N_DEV)
            sc_l = lax.rem(p + s, N_DEV)
            r = pltpu.make_async_remote_copy(
                src_ref=pbuf.at[w, pl.ds(sc_r * CHUNK, HALF)],
                dst_ref=recv_r.at[s, w],
                send_sem=send_sems.at[sem(0, 0, w, s)],
                recv_sem=recv_sems.at[sem(0, 0, w, s)],
                device_id=(right,),
                device_id_type=pl.DeviceIdType.MESH,
            )
            l = pltpu.make_async_remote_copy(
                src_ref=pbuf.at[w, pl.ds(sc_l * CHUNK + HALF, HALF)],
                dst_ref=recv_l.at[s, w],
                send_sem=send_sems.at[sem(0, 1, w, s)],
                recv_sem=recv_sems.at[sem(0, 1, w, s)],
                device_id=(left,),
                device_id_type=pl.DeviceIdType.MESH,
            )
            return r, l

        def ag_pair(w, s):
            gc_r = lax.rem(p + 1 - s + N_DEV, N_DEV)
            gc_l = lax.rem(p - 1 + s + N_DEV, N_DEV)
            r = pltpu.make_async_remote_copy(
                src_ref=pbuf.at[w, pl.ds(gc_r * CHUNK, HALF)],
                dst_ref=pbuf.at[w, pl.ds(gc_r * CHUNK, HALF)],
                send_sem=send_sems.at[sem(1, 0, w, s)],
                recv_sem=recv_sems.at[sem(1, 0, w, s)],
                device_id=(right,),
                device_id_type=pl.DeviceIdType.MESH,
            )
            l = pltpu.make_async_remote_copy(
                src_ref=pbuf.at[w, pl.ds(gc_l * CHUNK + HALF, HALF)],
                dst_ref=pbuf.at[w, pl.ds(gc_l * CHUNK + HALF, HALF)],
                send_sem=send_sems.at[sem(1, 1, w, s)],
                recv_sem=recv_sems.at[sem(1, 1, w, s)],
                device_id=(left,),
                device_id_type=pl.DeviceIdType.MESH,
            )
            return r, l

        def add_r(s, w):
            ac = lax.rem(p - 1 - s + N_DEV, N_DEV)
            pbuf[w, pl.ds(ac * CHUNK, HALF), :] += recv_r[s, w]

        def add_l(s, w):
            ac = lax.rem(p + 1 + s, N_DEV)
            pbuf[w, pl.ds(ac * CHUNK + HALF, HALF), :] += recv_l[s, w]

        def emit(c, rowhalf, w):
            rows = c * CHUNK + rowhalf * HALF
            out_ref[pl.ds(rows, HALF), w * WCOL:(w + 1) * WCOL] = (
                pbuf[w, pl.ds(rows, HALF), :].astype(jnp.float32))

        cp1 = lax.rem(p + 1, N_DEV)
        cm1 = lax.rem(p - 1 + N_DEV, N_DEV)
        cp2 = lax.rem(p + 2, N_DEV)

        ps_p = routing(p)
        compute_half(p, 0, ps_p, add_shared=True)

        barrier = pltpu.get_barrier_semaphore()
        for nbr in [left, right]:
            pl.semaphore_signal(barrier, inc=1, device_id=(nbr,),
                                device_id_type=pl.DeviceIdType.MESH)
        pl.semaphore_wait(barrier, 2)

        r0w0, l0w0 = rs_pair(0, 0)
        r0w0.start()
        l0w0.start()
        compute_half(p, 1, ps_p, add_shared=True)
        r0w1, l0w1 = rs_pair(1, 0)
        r0w1.start()
        l0w1.start()

        ps_p1 = routing(cp1)
        ps_m1 = routing(cm1)
        compute_half(cp1, 0, ps_p1)
        compute_half(cm1, 0, ps_m1)

        r0w0.wait()
        add_r(0, 0)
        r1w0, l1w0 = rs_pair(0, 1)
        r1w0.start()
        l0w0.wait()
        add_l(0, 0)
        l1w0.start()

        compute_half(cp1, 1, ps_p1)
        compute_half(cm1, 1, ps_m1)

        r0w1.wait()
        add_r(0, 1)
        r1w1, l1w1 = rs_pair(1, 1)
        r1w1.start()
        l0w1.wait()
        add_l(0, 1)
        l1w1.start()

        ps_p2 = routing(cp2)
        compute_half(cp2, 0, ps_p2)

        r1w0.wait()
        add_r(1, 0)
        r2w0, l2w0 = rs_pair(0, 2)
        r2w0.start()
        l1w0.wait()
        add_l(1, 0)
        l2w0.start()

        compute_half(cp2, 1, ps_p2)

        r1w1.wait()
        add_r(1, 1)
        r2w1, l2w1 = rs_pair(1, 2)
        r2w1.start()
        l1w1.wait()
        add_l(1, 1)
        l2w1.start()

        r2w0.wait()
        add_r(2, 0)
        agr0w0, agl0w0 = ag_pair(0, 0)
        agr0w0.start()
        l2w0.wait()
        add_l(2, 0)
        agl0w0.start()
        r2w1.wait()
        add_r(2, 1)
        agr0w1, agl0w1 = ag_pair(1, 0)
        agr0w1.start()
        l2w1.wait()
        add_l(2, 1)
        agl0w1.start()

        emit(cp1, 0, 0)
        emit(cp1, 0, 1)
        emit(cm1, 1, 0)
        emit(cm1, 1, 1)

        agr0w0.wait()
        agr1w0, agl1w0 = ag_pair(0, 1)
        agr1w0.start()
        agl0w0.wait()
        agl1w0.start()
        agr0w1.wait()
        agr1w1, agl1w1 = ag_pair(1, 1)
        agr1w1.start()
        agl0w1.wait()
        agl1w1.start()

        emit(p, 0, 0)
        emit(p, 0, 1)
        emit(p, 1, 0)
        emit(p, 1, 1)

        agr1w0.wait()
        agr2w0, agl2w0 = ag_pair(0, 2)
        agr2w0.start()
        agl1w0.wait()
        agl2w0.start()
        agr1w1.wait()
        agr2w1, agl2w1 = ag_pair(1, 2)
        agr2w1.start()
        agl1w1.wait()
        agl2w1.start()

        emit(cm1, 0, 0)
        emit(cm1, 0, 1)
        emit(cp1, 1, 0)
        emit(cp1, 1, 1)

        agr2w0.wait()
        agl2w0.wait()
        agr2w1.wait()
        agl2w1.wait()

        emit(cp2, 0, 0)
        emit(cp2, 0, 1)
        emit(cp2, 1, 0)
        emit(cp2, 1, 1)

    return pl.pallas_call(
        body,
        out_shape=jax.ShapeDtypeStruct((N_TOK, H), jnp.float32),
        in_specs=[pl.BlockSpec(memory_space=pltpu.VMEM)] * 5,
        out_specs=pl.BlockSpec(memory_space=pltpu.VMEM),
        scratch_shapes=[
            pltpu.VMEM((W, N_TOK, WCOL), jnp.bfloat16),
            pltpu.VMEM((N_DEV - 1, W, HALF, WCOL), jnp.bfloat16),
            pltpu.VMEM((N_DEV - 1, W, HALF, WCOL), jnp.bfloat16),
            pltpu.SemaphoreType.DMA((24,)),
            pltpu.SemaphoreType.DMA((24,)),
        ],
        compiler_params=pltpu.CompilerParams(
            collective_id=0,
            vmem_limit_bytes=100 * 1024 * 1024,
        ),
    )(x, router_W, route_idx, expert_W, shared_W)
